# baseline (speedup 1.0000x reference)
import functools

import jax
import jax.numpy as jnp
from jax.experimental import pallas as pl
from jax.experimental.pallas import tpu as pltpu

_THRESH = 0.5
_NEG_POS = 3
_B, _P, _C, _O = 32, 24564, 81, 16
_PB = 4096
_PPAD = 24576
_NPB = _PPAD // _PB
_SL = _PB // 128
_NCH = _PPAD // 1024


def _iou_one(gx1, gy1, gx2, gy2, px1, py1, px2, py2):
    ix1 = jnp.maximum(gx1, px1)
    iy1 = jnp.maximum(gy1, py1)
    ix2 = jnp.minimum(gx2, px2)
    iy2 = jnp.minimum(gy2, py2)
    inter = jnp.clip(ix2 - ix1, 0.0, None) * jnp.clip(iy2 - iy1, 0.0, None)
    area_g = (gx2 - gx1) * (gy2 - gy1)
    area_p = (px2 - px1) * (py2 - py1)
    return inter / (area_g + area_p - inter + 1e-10)


def _chunk_geom(prior_ref, c):
    pr = prior_ref[:, 8 * c:8 * (c + 1), :]
    cx, cy, w, h = pr[0], pr[1], pr[2], pr[3]
    return (cx - w / 2, cy - h / 2, cx + w / 2, cy + h / 2, cx, cy, w, h)


def _chunk_p2d(c):
    s = jax.lax.broadcasted_iota(jnp.int32, (8, 128), 0)
    l = jax.lax.broadcasted_iota(jnp.int32, (8, 128), 1)
    return c * 1024 + s * 128 + l


def _match_body(prior_ref, tgt_ref, loc_ref, tc_ref, stats_ref):
    b = pl.program_id(0)
    gts = [[tgt_ref[b, o, i] for i in range(5)] for o in range(_O)]

    vmax = [jnp.full((8, 128), -1.0, jnp.float32) for _ in range(_O)]
    varg = [jnp.zeros((8, 128), jnp.int32) for _ in range(_O)]
    for c in range(_NCH):
        px1, py1, px2, py2, _, _, _, _ = _chunk_geom(prior_ref, c)
        p2d = _chunk_p2d(c)
        for o in range(_O):
            g = gts[o]
            iou = _iou_one(g[0], g[1], g[2], g[3], px1, py1, px2, py2)
            upd = iou > vmax[o]
            vmax[o] = jnp.where(upd, iou, vmax[o])
            varg[o] = jnp.where(upd, p2d, varg[o])
    bpi = []
    for o in range(_O):
        m = jnp.max(vmax[o])
        bpi.append(jnp.min(jnp.where(vmax[o] == m, varg[o], _PPAD)))

    locacc = jnp.zeros((8, 128), jnp.float32)
    cntacc = jnp.zeros((8, 128), jnp.float32)
    for c in range(_NCH):
        px1, py1, px2, py2, cx, cy, w, h = _chunk_geom(prior_ref, c)
        p2d = _chunk_p2d(c)
        bgv = jnp.full((8, 128), -1.0, jnp.float32)
        bga = jnp.zeros((8, 128), jnp.int32)
        forced = jnp.full((8, 128), -1, jnp.int32)
        for o in range(_O):
            g = gts[o]
            iou = _iou_one(g[0], g[1], g[2], g[3], px1, py1, px2, py2)
            upd = iou > bgv
            bgv = jnp.where(upd, iou, bgv)
            bga = jnp.where(upd, o, bga)
            forced = jnp.where(p2d == bpi[o], o, forced)
        truth = jnp.where(forced >= 0, forced,
                          jnp.where(bgv >= _THRESH, bga, -1))
        mask = truth != -1
        safe = jnp.where(mask, truth, 0)

        mx1 = jnp.zeros((8, 128), jnp.float32)
        my1 = jnp.zeros((8, 128), jnp.float32)
        mx2 = jnp.zeros((8, 128), jnp.float32)
        my2 = jnp.zeros((8, 128), jnp.float32)
        mlab = jnp.zeros((8, 128), jnp.float32)
        for o in range(_O):
            sel = safe == o
            g = gts[o]
            mx1 = jnp.where(sel, g[0], mx1)
            my1 = jnp.where(sel, g[1], my1)
            mx2 = jnp.where(sel, g[2], mx2)
            my2 = jnp.where(sel, g[3], my2)
            mlab = jnp.where(sel, g[4], mlab)

        enc = (
            ((mx1 + mx2) / 2 - cx) / (w * 0.1),
            ((my1 + my2) / 2 - cy) / (h * 0.1),
            jnp.log((mx2 - mx1) / w + 1e-05) / 0.2,
            jnp.log((my2 - my1) / h + 1e-05) / 0.2,
        )
        ssum = jnp.zeros((8, 128), jnp.float32)
        for k in range(4):
            d = loc_ref[0, k, 8 * c:8 * (c + 1), :] - enc[k]
            ad = jnp.abs(d)
            ssum = ssum + jnp.where(ad < 1.0, 0.5 * d * d, ad - 0.5)
        locacc = locacc + jnp.where(mask, ssum, 0.0)

        tc = jnp.where(mask, mlab.astype(jnp.int32), 0)
        tc_ref[0, 8 * c:8 * (c + 1), :] = tc
        cntacc = cntacc + (tc > 0).astype(jnp.float32)

    locsum = jnp.sum(locacc)
    posn = jnp.sum(cntacc)
    s = jax.lax.broadcasted_iota(jnp.int32, (8, 128), 0)
    l = jax.lax.broadcasted_iota(jnp.int32, (8, 128), 1)
    stats_ref[0] = jnp.where((s == 0) & (l == 0), locsum,
                             jnp.where((s == 0) & (l == 1), posn, 0.0))


def _conf_body(tc_ref, conf_ref, ce_ref, posce_ref):
    j = pl.program_id(1)
    lse_acc = jnp.zeros((8, 128), jnp.float32)
    for c in range(_SL // 8):
        x3 = conf_ref[0, 1024 * c:1024 * (c + 1), :].reshape(8, 128, _C)
        lse = jnp.log(jnp.sum(jnp.exp(x3), axis=2))
        tc = tc_ref[0, 0, 8 * c:8 * (c + 1), :]
        posm = tc > 0
        valid = (j * _PB + _chunk_p2d(c)) < _P
        ce_ref[0, 0, 8 * c:8 * (c + 1), :] = jnp.where(
            valid & ~posm, lse - x3[:, :, 0], 0.0)
        cvec = jax.lax.broadcasted_iota(jnp.int32, (8, 128, _C), 2)
        m3 = (cvec == tc[:, :, None]) & (cvec > 0)
        pick_pos = jnp.sum(jnp.where(m3, x3, 0.0), axis=2)
        lse_acc = lse_acc + jnp.where(posm, lse, 0.0) - pick_pos
    acc = lse_acc

    @pl.when(j == 0)
    def _():
        posce_ref[0] = acc

    @pl.when(j != 0)
    def _():
        posce_ref[0] += acc


def _mine_body(kv_ref, ce_ref, out_ref):
    rows = ce_ref[...]
    bits = jax.lax.bitcast_convert_type(rows, jnp.int32)
    kv = kv_ref[:, 0]

    def step(i, cand):
        trial = cand + (jnp.int32(1) << (30 - i))
        cnt = jnp.sum((bits >= trial[:, None, None]).astype(jnp.int32),
                      axis=(1, 2))
        return jnp.where(cnt >= kv, trial, cand)

    t = jax.lax.fori_loop(0, 31, step, jnp.zeros((_B,), jnp.int32))
    gt = bits > t[:, None, None]
    cgt = jnp.sum(gt.astype(jnp.int32), axis=(1, 2))
    ties = (kv - cgt).astype(jnp.float32)
    s = jnp.sum(jnp.where(gt, rows, 0.0), axis=(1, 2))
    tval = jax.lax.bitcast_convert_type(t, jnp.float32)
    out_ref[...] = jnp.broadcast_to((s + ties * tval)[:, None], (_B, 128))


def _pallas_stages(pred_loc, pred_conf, targets, priors):
    priors4 = jnp.zeros((4, _PPAD), jnp.float32).at[:, :_P].set(priors.T)
    priors4 = priors4.reshape(4, _PPAD // 128, 128)
    ploc_t = jnp.zeros((_B, 4, _PPAD), jnp.float32).at[:, :, :_P].set(
        jnp.transpose(pred_loc, (0, 2, 1))).reshape(_B, 4, _PPAD // 128, 128)

    tc_map, stats = pl.pallas_call(
        _match_body,
        grid=(_B,),
        in_specs=[
            pl.BlockSpec((4, _PPAD // 128, 128), lambda b: (0, 0, 0)),
            pl.BlockSpec(memory_space=pltpu.SMEM),
            pl.BlockSpec((1, 4, _PPAD // 128, 128), lambda b: (b, 0, 0, 0)),
        ],
        out_specs=[
            pl.BlockSpec((1, _PPAD // 128, 128), lambda b: (b, 0, 0)),
            pl.BlockSpec((1, 8, 128), lambda b: (b, 0, 0)),
        ],
        out_shape=[
            jax.ShapeDtypeStruct((_B, _PPAD // 128, 128), jnp.int32),
            jax.ShapeDtypeStruct((_B, 8, 128), jnp.float32),
        ],
        compiler_params=pltpu.CompilerParams(
            dimension_semantics=("arbitrary",)),
    )(priors4, targets, ploc_t)

    ce, posce = pl.pallas_call(
        _conf_body,
        grid=(_B, _NPB),
        in_specs=[
            pl.BlockSpec((1, 1, _SL, 128), lambda b, j: (b, j, 0, 0)),
            pl.BlockSpec((1, _PB, _C), lambda b, j: (b, j, 0)),
        ],
        out_specs=[
            pl.BlockSpec((1, 1, _SL, 128), lambda b, j: (b, j, 0, 0)),
            pl.BlockSpec((1, 8, 128), lambda b, j: (b, 0, 0)),
        ],
        out_shape=[
            jax.ShapeDtypeStruct((_B, _NPB, _SL, 128), jnp.float32),
            jax.ShapeDtypeStruct((_B, 8, 128), jnp.float32),
        ],
        compiler_params=pltpu.CompilerParams(
            dimension_semantics=("arbitrary", "arbitrary")),
    )(tc_map.reshape(_B, _NPB, _SL, 128), pred_conf)

    pos_num = jnp.round(stats[:, 0, 1]).astype(jnp.int32)
    kvec = jnp.minimum(_NEG_POS * pos_num, _P - 1)

    neg = pl.pallas_call(
        _mine_body,
        grid=(1,),
        in_specs=[
            pl.BlockSpec((_B, 128), lambda i: (0, 0)),
            pl.BlockSpec((_B, _PPAD // 128, 128), lambda i: (0, 0, 0)),
        ],
        out_specs=pl.BlockSpec((_B, 128), lambda i: (0, 0)),
        out_shape=jax.ShapeDtypeStruct((_B, 128), jnp.float32),
        compiler_params=pltpu.CompilerParams(
            dimension_semantics=("arbitrary",)),
    )(jnp.broadcast_to(kvec[:, None], (_B, 128)),
      ce.reshape(_B, _PPAD // 128, 128))

    loc_loss = jnp.sum(stats[:, 0, 0]) / _B
    denom = jnp.maximum(jnp.sum(pos_num + kvec).astype(jnp.float32), 1.0)
    conf_loss = (jnp.sum(posce) + jnp.sum(neg[:, 0])) / denom / _B
    return loc_loss, conf_loss


def kernel(pred_loc, pred_conf, targets, priors):
    return _pallas_stages(pred_loc, pred_conf, targets, priors)


# speedup vs baseline: 1.1188x; 1.1188x over previous
import functools

import jax
import jax.numpy as jnp
from jax.experimental import pallas as pl
from jax.experimental.pallas import tpu as pltpu

_THRESH = 0.5
_NEG_POS = 3
_B, _P, _C, _O = 32, 24564, 81, 16
_PB = 4096
_PPAD = 24576
_NPB = _PPAD // _PB
_SL = _PB // 128
_NCH = _PPAD // 1024


def _iou_one(gx1, gy1, gx2, gy2, px1, py1, px2, py2):
    ix1 = jnp.maximum(gx1, px1)
    iy1 = jnp.maximum(gy1, py1)
    ix2 = jnp.minimum(gx2, px2)
    iy2 = jnp.minimum(gy2, py2)
    inter = jnp.clip(ix2 - ix1, 0.0, None) * jnp.clip(iy2 - iy1, 0.0, None)
    area_g = (gx2 - gx1) * (gy2 - gy1)
    area_p = (px2 - px1) * (py2 - py1)
    return inter / (area_g + area_p - inter + 1e-10)


def _chunk_geom(prior_ref, c):
    pr = prior_ref[:, 8 * c:8 * (c + 1), :]
    cx, cy, w, h = pr[0], pr[1], pr[2], pr[3]
    return (cx - w / 2, cy - h / 2, cx + w / 2, cy + h / 2, cx, cy, w, h)


def _chunk_p2d(c):
    s = jax.lax.broadcasted_iota(jnp.int32, (8, 128), 0)
    l = jax.lax.broadcasted_iota(jnp.int32, (8, 128), 1)
    return c * 1024 + s * 128 + l


def _match_body(prior_ref, tgt_ref, loc_ref, tc_ref, stats_ref):
    b = pl.program_id(0)
    gts = [[tgt_ref[b, o, i] for i in range(5)] for o in range(_O)]

    vmax = [jnp.full((8, 128), -1.0, jnp.float32) for _ in range(_O)]
    varg = [jnp.zeros((8, 128), jnp.int32) for _ in range(_O)]
    for c in range(_NCH):
        px1, py1, px2, py2, _, _, _, _ = _chunk_geom(prior_ref, c)
        p2d = _chunk_p2d(c)
        for o in range(_O):
            g = gts[o]
            iou = _iou_one(g[0], g[1], g[2], g[3], px1, py1, px2, py2)
            upd = iou > vmax[o]
            vmax[o] = jnp.where(upd, iou, vmax[o])
            varg[o] = jnp.where(upd, p2d, varg[o])
    bpi = []
    for o in range(_O):
        m = jnp.max(vmax[o])
        bpi.append(jnp.min(jnp.where(vmax[o] == m, varg[o], _PPAD)))

    locacc = jnp.zeros((8, 128), jnp.float32)
    cntacc = jnp.zeros((8, 128), jnp.float32)
    for c in range(_NCH):
        px1, py1, px2, py2, cx, cy, w, h = _chunk_geom(prior_ref, c)
        p2d = _chunk_p2d(c)
        bgv = jnp.full((8, 128), -1.0, jnp.float32)
        bga = jnp.zeros((8, 128), jnp.int32)
        forced = jnp.full((8, 128), -1, jnp.int32)
        for o in range(_O):
            g = gts[o]
            iou = _iou_one(g[0], g[1], g[2], g[3], px1, py1, px2, py2)
            upd = iou > bgv
            bgv = jnp.where(upd, iou, bgv)
            bga = jnp.where(upd, o, bga)
            forced = jnp.where(p2d == bpi[o], o, forced)
        truth = jnp.where(forced >= 0, forced,
                          jnp.where(bgv >= _THRESH, bga, -1))
        mask = truth != -1
        safe = jnp.where(mask, truth, 0)

        mx1 = jnp.zeros((8, 128), jnp.float32)
        my1 = jnp.zeros((8, 128), jnp.float32)
        mx2 = jnp.zeros((8, 128), jnp.float32)
        my2 = jnp.zeros((8, 128), jnp.float32)
        mlab = jnp.zeros((8, 128), jnp.float32)
        for o in range(_O):
            sel = safe == o
            g = gts[o]
            mx1 = jnp.where(sel, g[0], mx1)
            my1 = jnp.where(sel, g[1], my1)
            mx2 = jnp.where(sel, g[2], mx2)
            my2 = jnp.where(sel, g[3], my2)
            mlab = jnp.where(sel, g[4], mlab)

        enc = (
            ((mx1 + mx2) / 2 - cx) / (w * 0.1),
            ((my1 + my2) / 2 - cy) / (h * 0.1),
            jnp.log((mx2 - mx1) / w + 1e-05) / 0.2,
            jnp.log((my2 - my1) / h + 1e-05) / 0.2,
        )
        ssum = jnp.zeros((8, 128), jnp.float32)
        for k in range(4):
            d = loc_ref[0, k, 8 * c:8 * (c + 1), :] - enc[k]
            ad = jnp.abs(d)
            ssum = ssum + jnp.where(ad < 1.0, 0.5 * d * d, ad - 0.5)
        locacc = locacc + jnp.where(mask, ssum, 0.0)

        tc = jnp.where(mask, mlab.astype(jnp.int32), 0)
        tc_ref[0, 8 * c:8 * (c + 1), :] = tc
        cntacc = cntacc + (tc > 0).astype(jnp.float32)

    locsum = jnp.sum(locacc)
    posn = jnp.sum(cntacc)
    s = jax.lax.broadcasted_iota(jnp.int32, (8, 128), 0)
    l = jax.lax.broadcasted_iota(jnp.int32, (8, 128), 1)
    stats_ref[0] = jnp.where((s == 0) & (l == 0), locsum,
                             jnp.where((s == 0) & (l == 1), posn, 0.0))


def _conf_body(tc_ref, conf_ref, ce_ref, posce_ref):
    j = pl.program_id(1)
    acc = jnp.zeros((8, 128), jnp.float32)
    for c in range(_SL // 8):
        x3 = conf_ref[0, 1024 * c:1024 * (c + 1), :].reshape(8, 128, _C)
        lse = jnp.log(jnp.sum(jnp.exp(x3), axis=2))
        tc = tc_ref[0, 0, 8 * c:8 * (c + 1), :]
        cvec = jax.lax.broadcasted_iota(jnp.int32, (8, 128, _C), 2)
        picked = jnp.sum(jnp.where(cvec == tc[:, :, None], x3, 0.0), axis=2)
        ce = lse - picked
        posm = tc > 0
        valid = (j * _PB + _chunk_p2d(c)) < _P
        ce_ref[0, 0, 8 * c:8 * (c + 1), :] = jnp.where(
            valid & ~posm, ce, 0.0)
        acc = acc + jnp.where(posm, ce, 0.0)

    @pl.when(j == 0)
    def _():
        posce_ref[0] = acc

    @pl.when(j != 0)
    def _():
        posce_ref[0] += acc


def _mine_body(kv_ref, ce_ref, out_ref):
    rows = ce_ref[...]
    bits = jax.lax.bitcast_convert_type(rows, jnp.int32)
    kv = kv_ref[:, 0]

    def step(i, cand):
        trial = cand + (jnp.int32(1) << (30 - i))
        cnt = jnp.sum((bits >= trial[:, None, None]).astype(jnp.int32),
                      axis=(1, 2))
        return jnp.where(cnt >= kv, trial, cand)

    t = jax.lax.fori_loop(0, 31, step, jnp.zeros((_B,), jnp.int32))
    gt = bits > t[:, None, None]
    cgt = jnp.sum(gt.astype(jnp.int32), axis=(1, 2))
    ties = (kv - cgt).astype(jnp.float32)
    s = jnp.sum(jnp.where(gt, rows, 0.0), axis=(1, 2))
    tval = jax.lax.bitcast_convert_type(t, jnp.float32)
    out_ref[...] = jnp.broadcast_to((s + ties * tval)[:, None], (_B, 128))


def _pallas_stages(pred_loc, pred_conf, targets, priors):
    priors4 = jnp.zeros((4, _PPAD), jnp.float32).at[:, :_P].set(priors.T)
    priors4 = priors4.reshape(4, _PPAD // 128, 128)
    ploc_t = jnp.zeros((_B, 4, _PPAD), jnp.float32).at[:, :, :_P].set(
        jnp.transpose(pred_loc, (0, 2, 1))).reshape(_B, 4, _PPAD // 128, 128)

    tc_map, stats = pl.pallas_call(
        _match_body,
        grid=(_B,),
        in_specs=[
            pl.BlockSpec((4, _PPAD // 128, 128), lambda b: (0, 0, 0)),
            pl.BlockSpec(memory_space=pltpu.SMEM),
            pl.BlockSpec((1, 4, _PPAD // 128, 128), lambda b: (b, 0, 0, 0)),
        ],
        out_specs=[
            pl.BlockSpec((1, _PPAD // 128, 128), lambda b: (b, 0, 0)),
            pl.BlockSpec((1, 8, 128), lambda b: (b, 0, 0)),
        ],
        out_shape=[
            jax.ShapeDtypeStruct((_B, _PPAD // 128, 128), jnp.int32),
            jax.ShapeDtypeStruct((_B, 8, 128), jnp.float32),
        ],
        compiler_params=pltpu.CompilerParams(
            dimension_semantics=("arbitrary",)),
    )(priors4, targets, ploc_t)

    ce, posce = pl.pallas_call(
        _conf_body,
        grid=(_B, _NPB),
        in_specs=[
            pl.BlockSpec((1, 1, _SL, 128), lambda b, j: (b, j, 0, 0)),
            pl.BlockSpec((1, _PB, _C), lambda b, j: (b, j, 0)),
        ],
        out_specs=[
            pl.BlockSpec((1, 1, _SL, 128), lambda b, j: (b, j, 0, 0)),
            pl.BlockSpec((1, 8, 128), lambda b, j: (b, 0, 0)),
        ],
        out_shape=[
            jax.ShapeDtypeStruct((_B, _NPB, _SL, 128), jnp.float32),
            jax.ShapeDtypeStruct((_B, 8, 128), jnp.float32),
        ],
        compiler_params=pltpu.CompilerParams(
            dimension_semantics=("arbitrary", "arbitrary")),
    )(tc_map.reshape(_B, _NPB, _SL, 128), pred_conf)

    pos_num = jnp.round(stats[:, 0, 1]).astype(jnp.int32)
    kvec = jnp.minimum(_NEG_POS * pos_num, _P - 1)

    neg = pl.pallas_call(
        _mine_body,
        grid=(1,),
        in_specs=[
            pl.BlockSpec((_B, 128), lambda i: (0, 0)),
            pl.BlockSpec((_B, _PPAD // 128, 128), lambda i: (0, 0, 0)),
        ],
        out_specs=pl.BlockSpec((_B, 128), lambda i: (0, 0)),
        out_shape=jax.ShapeDtypeStruct((_B, 128), jnp.float32),
        compiler_params=pltpu.CompilerParams(
            dimension_semantics=("arbitrary",)),
    )(jnp.broadcast_to(kvec[:, None], (_B, 128)),
      ce.reshape(_B, _PPAD // 128, 128))

    loc_loss = jnp.sum(stats[:, 0, 0]) / _B
    denom = jnp.maximum(jnp.sum(pos_num + kvec).astype(jnp.float32), 1.0)
    conf_loss = (jnp.sum(posce) + jnp.sum(neg[:, 0])) / denom / _B
    return loc_loss, conf_loss


def kernel(pred_loc, pred_conf, targets, priors):
    return _pallas_stages(pred_loc, pred_conf, targets, priors)
